# baseline (speedup 1.0000x reference)
import functools

import jax
import jax.numpy as jnp
from jax import lax
from jax.experimental import pallas as pl
from jax.experimental.pallas import tpu as pltpu
from jax.experimental.pallas import tpu_sc as plsc

B, S, H, I = 4, 2048, 768, 3072
N = B * S
NC, NS = 2, 16
NW = NC * NS
ROWS_PER_W = N // NW
CHUNK = 64
NCHUNK = ROWS_PER_W // CHUNK

TBLK = 512
NBLK = N // TBLK
NPOS = S // TBLK


def _sc_gather(table, idx3):
    mesh = plsc.VectorSubcoreMesh(core_axis_name="c", subcore_axis_name="s")

    @functools.partial(
        pl.kernel,
        mesh=mesh,
        out_type=jax.ShapeDtypeStruct((N, H), jnp.float32),
        scratch_types=[
            pltpu.VMEM((NCHUNK, CHUNK), jnp.int32),
            pltpu.VMEM((CHUNK, H), jnp.float32),
            pltpu.VMEM((CHUNK, H), jnp.float32),
            pltpu.SemaphoreType.DMA,
            pltpu.SemaphoreType.DMA,
        ],
    )
    def k(table_hbm, idx_hbm, out_hbm, idx_v, rows0, rows1, sem0, sem1):
        wid = lax.axis_index("s") * NC + lax.axis_index("c")
        base = wid * ROWS_PER_W
        pltpu.sync_copy(idx_hbm.at[wid], idx_v)
        bufs = (rows0, rows1)
        sems = (sem0, sem1)
        handles = [None] * NCHUNK
        handles[0] = pltpu.async_copy(table_hbm.at[idx_v.at[0]], bufs[0], sems[0])
        for c in range(NCHUNK):
            if c + 1 < NCHUNK:
                handles[c + 1] = pltpu.async_copy(
                    table_hbm.at[idx_v.at[c + 1]], bufs[(c + 1) % 2], sems[(c + 1) % 2])
            handles[c].wait()
            pltpu.sync_copy(bufs[c % 2], out_hbm.at[pl.ds(base + c * CHUNK, CHUNK)])

    return k(table, idx3)


def _ln_blk(x, g, b):
    m = jnp.mean(x, axis=-1, keepdims=True)
    v = jnp.mean((x - m) ** 2, axis=-1, keepdims=True)
    return (x - m) * lax.rsqrt(v + 1e-12) * g + b


def _tc_body(xg, tt, pos, tok, ln0g, ln0b,
             Wi0, bi0, Wo0, bo0, g0, b0,
             Wi1, bi1, Wo1, bo1, g1, b1, out, xmid):
    i = pl.program_id(0)
    par = i % 2
    off_w = pl.multiple_of(par * TBLK, TBLK)
    off_r = pl.multiple_of((1 - par) * TBLK, TBLK)

    x1 = xmid[pl.ds(off_r, TBLK), :]

    ttv = tt[...].astype(jnp.float32)
    tok0 = tok[0:1, :]
    tok1 = tok[1:2, :]
    x = xg[...] + tok0 + ttv * (tok1[...] - tok0) + pos[...]
    x = _ln_blk(x, ln0g[...], ln0b[...])
    h = jax.nn.gelu(jnp.dot(x, Wi0[...], preferred_element_type=jnp.float32)
                    + bi0[...])
    o = jnp.dot(h, Wo0[...], preferred_element_type=jnp.float32) + bo0[...]
    xmid[pl.ds(off_w, TBLK), :] = _ln_blk(o + x, g0[...], b0[...])

    h1 = jax.nn.gelu(jnp.dot(x1, Wi1[...], preferred_element_type=jnp.float32)
                     + bi1[...])
    o1 = jnp.dot(h1, Wo1[...], preferred_element_type=jnp.float32) + bo1[...]
    out[...] = _ln_blk(o1 + x1, g1[...], b1[...])


def _row(v):
    return v.reshape(1, -1)


def kernel(input_ids, token_type_ids, word_emb, tok_emb, pos_emb, ln0_g, ln0_b,
           l0_Wi, l0_bi, l0_Wo, l0_bo, l0_g, l0_b,
           l1_Wi, l1_bi, l1_Wo, l1_bo, l1_g, l1_b):
    ids = input_ids.reshape(-1).astype(jnp.int32).reshape(NW, NCHUNK, CHUNK)
    gathered = _sc_gather(word_emb, ids)
    tt = token_type_ids.astype(jnp.int32).reshape(N, 1)

    whole = lambda shape: pl.BlockSpec(shape, lambda i: (0, 0))
    last = NBLK - 1
    in_specs = [
        pl.BlockSpec((TBLK, H), lambda i: (jnp.minimum(i, last), 0)),
        pl.BlockSpec((TBLK, 1), lambda i: (jnp.minimum(i, last), 0)),
        pl.BlockSpec((TBLK, H),
                     lambda i: (jnp.minimum(i, last) % NPOS, 0)),
        whole((2, H)),
        whole((1, H)), whole((1, H)),
        whole((H, I)), whole((1, I)), whole((I, H)), whole((1, H)),
        whole((1, H)), whole((1, H)),
        whole((H, I)), whole((1, I)), whole((I, H)), whole((1, H)),
        whole((1, H)), whole((1, H)),
    ]
    out = pl.pallas_call(
        _tc_body,
        grid=(NBLK + 1,),
        in_specs=in_specs,
        out_specs=pl.BlockSpec((TBLK, H), lambda i: (jnp.maximum(i - 1, 0), 0)),
        out_shape=jax.ShapeDtypeStruct((N, H), jnp.float32),
        scratch_shapes=[pltpu.VMEM((2 * TBLK, H), jnp.float32)],
        compiler_params=pltpu.CompilerParams(
            dimension_semantics=("arbitrary",),
        ),
    )(gathered, tt, pos_emb, tok_emb, _row(ln0_g), _row(ln0_b),
      l0_Wi, _row(l0_bi), l0_Wo, _row(l0_bo), _row(l0_g), _row(l0_b),
      l1_Wi, _row(l1_bi), l1_Wo, _row(l1_bo), _row(l1_g), _row(l1_b))
    return out.reshape(B, S, H)


# speedup vs baseline: 1.0531x; 1.0531x over previous
import functools

import jax
import jax.numpy as jnp
from jax import lax
from jax.experimental import pallas as pl
from jax.experimental.pallas import tpu as pltpu
from jax.experimental.pallas import tpu_sc as plsc

B, S, H, I = 4, 2048, 768, 3072
N = B * S
NC, NS = 2, 16
NW = NC * NS
ROWS_PER_W = N // NW
CHUNK = 64
NCHUNK = ROWS_PER_W // CHUNK

TBLK = 512
NBLK = N // TBLK
NPOS = S // TBLK


def _sc_gather(table, idx3):
    mesh = plsc.VectorSubcoreMesh(core_axis_name="c", subcore_axis_name="s")

    @functools.partial(
        pl.kernel,
        mesh=mesh,
        out_type=jax.ShapeDtypeStruct((N, H), jnp.float32),
        scratch_types=[
            pltpu.VMEM((NCHUNK, CHUNK), jnp.int32),
            pltpu.VMEM((CHUNK, H), jnp.float32),
            pltpu.VMEM((CHUNK, H), jnp.float32),
            pltpu.SemaphoreType.DMA,
            pltpu.SemaphoreType.DMA,
        ],
    )
    def k(table_hbm, idx_hbm, out_hbm, idx_v, rows0, rows1, sem0, sem1):
        wid = lax.axis_index("s") * NC + lax.axis_index("c")
        base = wid * ROWS_PER_W
        pltpu.sync_copy(idx_hbm.at[wid], idx_v)
        bufs = (rows0, rows1)
        sems = (sem0, sem1)
        handles = [None] * NCHUNK
        handles[0] = pltpu.async_copy(table_hbm.at[idx_v.at[0]], bufs[0], sems[0])
        for c in range(NCHUNK):
            if c + 1 < NCHUNK:
                handles[c + 1] = pltpu.async_copy(
                    table_hbm.at[idx_v.at[c + 1]], bufs[(c + 1) % 2], sems[(c + 1) % 2])
            handles[c].wait()
            pltpu.sync_copy(bufs[c % 2], out_hbm.at[pl.ds(base + c * CHUNK, CHUNK)])

    return k(table, idx3)


def _ln_blk(x, g, b):
    m = jnp.mean(x, axis=-1, keepdims=True)
    v = jnp.mean((x - m) ** 2, axis=-1, keepdims=True)
    return (x - m) * lax.rsqrt(v + 1e-12) * g + b


def _tc_body(xg, tt, pos, tok, ln0g, ln0b,
             Wi0, bi0, Wo0, bo0, g0, b0,
             Wi1, bi1, Wo1, bo1, g1, b1, out):
    ttv = tt[...].astype(jnp.float32)
    tok0 = tok[0:1, :]
    tok1 = tok[1:2, :]
    x = xg[...] + tok0 + ttv * (tok1[...] - tok0) + pos[...]
    x = _ln_blk(x, ln0g[...], ln0b[...])
    for (Wi, bi, Wo, bo, g, b) in ((Wi0, bi0, Wo0, bo0, g0, b0),
                                   (Wi1, bi1, Wo1, bo1, g1, b1)):
        h = jax.nn.gelu(jnp.dot(x, Wi[...], preferred_element_type=jnp.float32)
                        + bi[...])
        o = jnp.dot(h, Wo[...], preferred_element_type=jnp.float32) + bo[...]
        x = _ln_blk(o + x, g[...], b[...])
    out[...] = x


def _row(v):
    return v.reshape(1, -1)


def kernel(input_ids, token_type_ids, word_emb, tok_emb, pos_emb, ln0_g, ln0_b,
           l0_Wi, l0_bi, l0_Wo, l0_bo, l0_g, l0_b,
           l1_Wi, l1_bi, l1_Wo, l1_bo, l1_g, l1_b):
    ids = input_ids.reshape(-1).astype(jnp.int32).reshape(NW, NCHUNK, CHUNK)
    gathered = _sc_gather(word_emb, ids)
    tt = token_type_ids.astype(jnp.int32).reshape(N, 1)

    whole = lambda shape: pl.BlockSpec(shape, lambda i: (0, 0))
    in_specs = [
        pl.BlockSpec((TBLK, H), lambda i: (i, 0)),
        pl.BlockSpec((TBLK, 1), lambda i: (i, 0)),
        pl.BlockSpec((TBLK, H), lambda i: (i % NPOS, 0)),
        whole((2, H)),
        whole((1, H)), whole((1, H)),
        whole((H, I)), whole((1, I)), whole((I, H)), whole((1, H)),
        whole((1, H)), whole((1, H)),
        whole((H, I)), whole((1, I)), whole((I, H)), whole((1, H)),
        whole((1, H)), whole((1, H)),
    ]
    out = pl.pallas_call(
        _tc_body,
        grid=(NBLK,),
        in_specs=in_specs,
        out_specs=pl.BlockSpec((TBLK, H), lambda i: (i, 0)),
        out_shape=jax.ShapeDtypeStruct((N, H), jnp.float32),
        compiler_params=pltpu.CompilerParams(
            dimension_semantics=("parallel",),
        ),
    )(gathered, tt, pos_emb, tok_emb, _row(ln0_g), _row(ln0_b),
      l0_Wi, _row(l0_bi), l0_Wo, _row(l0_bo), _row(l0_g), _row(l0_b),
      l1_Wi, _row(l1_bi), l1_Wo, _row(l1_bo), _row(l1_g), _row(l1_b))
    return out.reshape(B, S, H)


# speedup vs baseline: 1.2893x; 1.2242x over previous
import functools

import jax
import jax.numpy as jnp
from jax import lax
from jax.experimental import pallas as pl
from jax.experimental.pallas import tpu as pltpu
from jax.experimental.pallas import tpu_sc as plsc

B, S, H, I = 4, 2048, 768, 3072
N = B * S
NC, NS = 2, 16
NW = NC * NS
ROWS_PER_W = N // NW
CHUNK = 64
NCHUNK = ROWS_PER_W // CHUNK

TBLK = 512
NBLK = N // TBLK
NPOS = S // TBLK


def _sc_gather(table, idx3):
    mesh = plsc.VectorSubcoreMesh(core_axis_name="c", subcore_axis_name="s")

    @functools.partial(
        pl.kernel,
        mesh=mesh,
        out_type=jax.ShapeDtypeStruct((N, H), jnp.float32),
        scratch_types=[
            pltpu.VMEM((NCHUNK, CHUNK), jnp.int32),
            pltpu.VMEM((CHUNK, H), jnp.float32),
            pltpu.VMEM((CHUNK, H), jnp.float32),
            pltpu.SemaphoreType.DMA,
            pltpu.SemaphoreType.DMA,
        ],
    )
    def k(table_hbm, idx_hbm, out_hbm, idx_v, rows0, rows1, sem0, sem1):
        wid = lax.axis_index("s") * NC + lax.axis_index("c")
        base = wid * ROWS_PER_W
        pltpu.sync_copy(idx_hbm.at[wid], idx_v)
        bufs = (rows0, rows1)
        sems = (sem0, sem1)
        handles = [None] * NCHUNK
        handles[0] = pltpu.async_copy(table_hbm.at[idx_v.at[0]], bufs[0], sems[0])
        for c in range(NCHUNK):
            if c + 1 < NCHUNK:
                handles[c + 1] = pltpu.async_copy(
                    table_hbm.at[idx_v.at[c + 1]], bufs[(c + 1) % 2], sems[(c + 1) % 2])
            handles[c].wait()
            pltpu.sync_copy(bufs[c % 2], out_hbm.at[pl.ds(base + c * CHUNK, CHUNK)])

    return k(table, idx3)


def _ln_blk(x, g, b):
    m = jnp.mean(x, axis=-1, keepdims=True)
    v = jnp.mean((x - m) ** 2, axis=-1, keepdims=True)
    return (x - m) * lax.rsqrt(v + 1e-12) * g + b


def _tc_body(xg, tt, pos, tok, ln0g, ln0b,
             Wi0, bi0, Wo0, bo0, g0, b0,
             Wi1, bi1, Wo1, bo1, g1, b1, out):
    ttv = tt[...].astype(jnp.float32)
    tok0 = tok[0:1, :]
    tok1 = tok[1:2, :]
    x = xg[...] + tok0 + ttv * (tok1[...] - tok0) + pos[...]
    x = _ln_blk(x, ln0g[...], ln0b[...])
    for (Wi, bi, Wo, bo, g, b) in ((Wi0, bi0, Wo0, bo0, g0, b0),
                                   (Wi1, bi1, Wo1, bo1, g1, b1)):
        y = jnp.dot(x, Wi[...], preferred_element_type=jnp.float32) + bi[...]
        h = jax.nn.gelu(y.astype(jnp.bfloat16))
        o = lax.dot_general(h, Wo[...], (((1,), (0,)), ((), ())),
                            preferred_element_type=jnp.float32) + bo[...]
        x = _ln_blk(o + x, g[...], b[...])
    out[...] = x


def _row(v):
    return v.reshape(1, -1)


def kernel(input_ids, token_type_ids, word_emb, tok_emb, pos_emb, ln0_g, ln0_b,
           l0_Wi, l0_bi, l0_Wo, l0_bo, l0_g, l0_b,
           l1_Wi, l1_bi, l1_Wo, l1_bo, l1_g, l1_b):
    ids = input_ids.reshape(-1).astype(jnp.int32).reshape(NW, NCHUNK, CHUNK)
    gathered = _sc_gather(word_emb, ids)
    tt = token_type_ids.astype(jnp.int32).reshape(N, 1)

    whole = lambda shape: pl.BlockSpec(shape, lambda i: (0, 0))
    in_specs = [
        pl.BlockSpec((TBLK, H), lambda i: (i, 0)),
        pl.BlockSpec((TBLK, 1), lambda i: (i, 0)),
        pl.BlockSpec((TBLK, H), lambda i: (i % NPOS, 0)),
        whole((2, H)),
        whole((1, H)), whole((1, H)),
        whole((H, I)), whole((1, I)), whole((I, H)), whole((1, H)),
        whole((1, H)), whole((1, H)),
        whole((H, I)), whole((1, I)), whole((I, H)), whole((1, H)),
        whole((1, H)), whole((1, H)),
    ]
    out = pl.pallas_call(
        _tc_body,
        grid=(NBLK,),
        in_specs=in_specs,
        out_specs=pl.BlockSpec((TBLK, H), lambda i: (i, 0)),
        out_shape=jax.ShapeDtypeStruct((N, H), jnp.float32),
        compiler_params=pltpu.CompilerParams(
            dimension_semantics=("arbitrary",),
        ),
    )(gathered, tt, pos_emb, tok_emb, _row(ln0_g), _row(ln0_b),
      l0_Wi, _row(l0_bi), l0_Wo, _row(l0_bo), _row(l0_g), _row(l0_b),
      l1_Wi, _row(l1_bi), l1_Wo, _row(l1_bo), _row(l1_g), _row(l1_b))
    return out.reshape(B, S, H)
